# baseline (speedup 1.0000x reference)
import jax
import jax.numpy as jnp
from jax import lax
from jax.experimental import pallas as pl
from jax.experimental.pallas import tpu as pltpu
from jax.experimental.pallas import tpu_sc as plsc

E = 1_000_000
R = 1_000
D = 64
B = 16384
LMBDA = 0.1

NC = 2
NS = 16
NW = NC * NS
BW = B // NW
CH = 32
NCHUNK = BW // CH
FIRE = 8


def _row_scalar(idx_ref, p):
    return (idx_ref[pl.ds(p, 16)])[0]


def _sc_body(h_hbm, t_hbm, r_hbm, ent_re, ent_im, rel_re, rel_im,
             res_out, sq_out,
             idx_h, idx_t, idx_r,
             hre_v, him_v, tre_v, tim_v, rre_v, rim_v,
             res_v, sq_v, stage_v, sem):
    wid = lax.axis_index("s") * NC + lax.axis_index("c")
    base = wid * BW
    pltpu.sync_copy(h_hbm.at[pl.ds(base, BW)], idx_h.at[pl.ds(0, BW)])
    pltpu.sync_copy(t_hbm.at[pl.ds(base, BW)], idx_t.at[pl.ds(0, BW)])
    pltpu.sync_copy(r_hbm.at[pl.ds(base, BW)], idx_r.at[pl.ds(0, BW)])

    def chunk_body(c, sq_acc):
        for b in range(CH // FIRE):
            cps = []
            for jj in range(b * FIRE, (b + 1) * FIRE):
                p = c * CH + jj
                rh = _row_scalar(idx_h, p) >> 1
                rt = _row_scalar(idx_t, p) >> 1
                rr = _row_scalar(idx_r, p) >> 1
                cps += [
                    pltpu.async_copy(ent_re.at[pl.ds(rh, 1), :], hre_v.at[jj], sem),
                    pltpu.async_copy(ent_im.at[pl.ds(rh, 1), :], him_v.at[jj], sem),
                    pltpu.async_copy(ent_re.at[pl.ds(rt, 1), :], tre_v.at[jj], sem),
                    pltpu.async_copy(ent_im.at[pl.ds(rt, 1), :], tim_v.at[jj], sem),
                    pltpu.async_copy(rel_re.at[pl.ds(rr, 1), :], rre_v.at[jj], sem),
                    pltpu.async_copy(rel_im.at[pl.ds(rr, 1), :], rim_v.at[jj], sem),
                ]
            for cp in cps:
                cp.wait()

        def group_body(g, sq_in):
            def row_body(k, sq2):
                j = g * 16 + k
                p = c * CH + j
                sh = (_row_scalar(idx_h, p) & 1) * 64
                st = (_row_scalar(idx_t, p) & 1) * 64
                sr = (_row_scalar(idx_r, p) & 1) * 64
                e_acc = jnp.zeros((16,), jnp.float32)
                for blk in range(D // 16):
                    o = blk * 16
                    hre = hre_v[j, 0, pl.ds(sh + o, 16)]
                    him = him_v[j, 0, pl.ds(sh + o, 16)]
                    tre = tre_v[j, 0, pl.ds(st + o, 16)]
                    tim = tim_v[j, 0, pl.ds(st + o, 16)]
                    rre = rre_v[j, 0, pl.ds(sr + o, 16)]
                    rim = rim_v[j, 0, pl.ds(sr + o, 16)]
                    e_acc = (e_acc + hre * (tre * rre + tim * rim)
                             + him * (tim * rre - tre * rim))
                    sq2 = (sq2 + hre * hre + him * him + tre * tre
                           + tim * tim + rre * rre + rim * rim)
                stage_v[pl.ds(k * 16, 16)] = e_acc
                return sq2

            sq3 = lax.fori_loop(0, 16, row_body, sq_in)
            rows16 = lax.iota(jnp.int32, 16) * 16
            tot = jnp.zeros((16,), jnp.float32)
            for d in range(16):
                tot = tot + plsc.load_gather(stage_v, [rows16 + d])
            res_v[pl.ds(c * CH + g * 16, 16)] = tot
            return sq3

        return lax.fori_loop(0, CH // 16, group_body, sq_acc)

    sq_acc = lax.fori_loop(0, NCHUNK, chunk_body, jnp.zeros((16,), jnp.float32))
    sq_v[...] = sq_acc
    pltpu.sync_copy(res_v, res_out.at[pl.ds(base, BW)])
    pltpu.sync_copy(sq_v, sq_out.at[pl.ds(wid * 16, 16)])


_sc_call = pl.kernel(
    _sc_body,
    out_type=[jax.ShapeDtypeStruct((B,), jnp.float32),
              jax.ShapeDtypeStruct((NW * 16,), jnp.float32)],
    mesh=plsc.VectorSubcoreMesh(core_axis_name="c", subcore_axis_name="s"),
    compiler_params=pltpu.CompilerParams(needs_layout_passes=False,
                                         use_tc_tiling_on_sc=True),
    scratch_types=[
        pltpu.VMEM((BW + 16,), jnp.int32),
        pltpu.VMEM((BW + 16,), jnp.int32),
        pltpu.VMEM((BW + 16,), jnp.int32),
        pltpu.VMEM((CH, 1, 2 * D), jnp.float32),
        pltpu.VMEM((CH, 1, 2 * D), jnp.float32),
        pltpu.VMEM((CH, 1, 2 * D), jnp.float32),
        pltpu.VMEM((CH, 1, 2 * D), jnp.float32),
        pltpu.VMEM((CH, 1, 2 * D), jnp.float32),
        pltpu.VMEM((CH, 1, 2 * D), jnp.float32),
        pltpu.VMEM((BW,), jnp.float32),
        pltpu.VMEM((16,), jnp.float32),
        pltpu.VMEM((256,), jnp.float32),
        pltpu.SemaphoreType.DMA,
    ],
)


def _tc_body(res_ref, y_ref, sq_ref, out_ref):
    res = res_ref[...]
    y = y_ref[...]
    loss = jnp.sum(jax.nn.softplus(-y * res)) / B
    regul = jnp.sum(sq_ref[...]) / (B * D)
    out_ref[0, 0] = loss + LMBDA * regul


def kernel(h, t, r, y, ent_re, ent_im, rel_re, rel_im):
    h32 = h.astype(jnp.int32)
    t32 = t.astype(jnp.int32)
    r32 = r.astype(jnp.int32)
    res, sq = _sc_call(h32, t32, r32,
                       ent_re.reshape(E // 2, 2 * D),
                       ent_im.reshape(E // 2, 2 * D),
                       rel_re.reshape(R // 2, 2 * D),
                       rel_im.reshape(R // 2, 2 * D))
    out = pl.pallas_call(
        _tc_body,
        out_shape=jax.ShapeDtypeStruct((1, 1), jnp.float32),
        out_specs=pl.BlockSpec(memory_space=pltpu.SMEM),
    )(res.reshape(128, 128), y.reshape(128, 128), sq.reshape(4, 128))
    return out[0, 0]


# speedup vs baseline: 1.5835x; 1.5835x over previous
import jax
import jax.numpy as jnp
from jax import lax
from jax.experimental import pallas as pl
from jax.experimental.pallas import tpu as pltpu
from jax.experimental.pallas import tpu_sc as plsc

E = 1_000_000
R = 1_000
D = 64
B = 16384
LMBDA = 0.1

NC = 2
NS = 16
NW = NC * NS
BW = B // NW
CH = 16
NCHUNK = BW // CH
N2 = NCHUNK // 2


def _row_scalar(idx_ref, p):
    return (idx_ref[pl.ds(p, 16)])[0]


def _sc_body(h_hbm, t_hbm, r_hbm, ent_re, ent_im, rel_re, rel_im,
             res_out, sq_out,
             idx_h, idx_t, idx_r,
             bufs_a, bufs_b,
             res_v, sq_v, stage_v, sem_a, sem_b):
    wid = lax.axis_index("s") * NC + lax.axis_index("c")
    base = wid * BW
    pltpu.sync_copy(h_hbm.at[pl.ds(base, BW)], idx_h.at[pl.ds(0, BW)])
    pltpu.sync_copy(t_hbm.at[pl.ds(base, BW)], idx_t.at[pl.ds(0, BW)])
    pltpu.sync_copy(r_hbm.at[pl.ds(base, BW)], idx_r.at[pl.ds(0, BW)])

    tables = (ent_re, ent_im, ent_re, ent_im, rel_re, rel_im)

    def issue(c, bufs, sem):
        for jj in range(CH):
            p = c * CH + jj
            rh = _row_scalar(idx_h, p)
            rt = _row_scalar(idx_t, p)
            rr = _row_scalar(idx_r, p)
            for q, (tbl, row) in enumerate(
                    zip(tables, (rh, rh, rt, rt, rr, rr))):
                pltpu.async_copy(tbl.at[pl.ds(row, 1), :],
                                 bufs[q].at[jj], sem)

    def drain(bufs, sem):
        for jj in range(CH):
            for q, tbl in enumerate(tables):
                pltpu.make_async_copy(tbl.at[pl.ds(0, 1), :],
                                      bufs[q].at[jj], sem).wait()

    def compute(c, bufs, sq_in):
        hre_v, him_v, tre_v, tim_v, rre_v, rim_v = bufs

        def row_body(k, sq2):
            e_acc = jnp.zeros((16,), jnp.float32)
            for blk in range(D // 16):
                sl = pl.ds(blk * 16, 16)
                hre = hre_v[k, 0, sl]
                him = him_v[k, 0, sl]
                tre = tre_v[k, 0, sl]
                tim = tim_v[k, 0, sl]
                rre = rre_v[k, 0, sl]
                rim = rim_v[k, 0, sl]
                e_acc = (e_acc + hre * (tre * rre + tim * rim)
                         + him * (tim * rre - tre * rim))
                sq2 = (sq2 + hre * hre + him * him + tre * tre
                       + tim * tim + rre * rre + rim * rim)
            stage_v[pl.ds(k * 16, 16)] = e_acc
            return sq2

        sq3 = lax.fori_loop(0, CH, row_body, sq_in)
        rows16 = lax.iota(jnp.int32, 16) * 16
        tot = jnp.zeros((16,), jnp.float32)
        for d in range(16):
            tot = tot + plsc.load_gather(stage_v, [rows16 + d])
        res_v[pl.ds(c * CH, 16)] = tot
        return sq3

    issue(0, bufs_a, sem_a)

    def pair_body(i, sq_acc):
        issue(2 * i + 1, bufs_b, sem_b)
        drain(bufs_a, sem_a)
        sq_acc = compute(2 * i, bufs_a, sq_acc)

        @pl.when(i < N2 - 1)
        def _():
            issue(2 * i + 2, bufs_a, sem_a)

        drain(bufs_b, sem_b)
        return compute(2 * i + 1, bufs_b, sq_acc)

    sq_acc = lax.fori_loop(0, N2, pair_body, jnp.zeros((16,), jnp.float32))
    sq_v[...] = sq_acc
    pltpu.sync_copy(res_v, res_out.at[pl.ds(base, BW)])
    pltpu.sync_copy(sq_v, sq_out.at[pl.ds(wid * 16, 16)])


_GATHER_BUFS = [pltpu.VMEM((CH, 1, D), jnp.float32)] * 6

_sc_call = pl.kernel(
    _sc_body,
    out_type=[jax.ShapeDtypeStruct((B,), jnp.float32),
              jax.ShapeDtypeStruct((NW * 16,), jnp.float32)],
    mesh=plsc.VectorSubcoreMesh(core_axis_name="c", subcore_axis_name="s"),
    compiler_params=pltpu.CompilerParams(needs_layout_passes=False,
                                         use_tc_tiling_on_sc=True),
    scratch_types=[
        pltpu.VMEM((BW + 16,), jnp.int32),
        pltpu.VMEM((BW + 16,), jnp.int32),
        pltpu.VMEM((BW + 16,), jnp.int32),
        list(_GATHER_BUFS),
        list(_GATHER_BUFS),
        pltpu.VMEM((BW,), jnp.float32),
        pltpu.VMEM((16,), jnp.float32),
        pltpu.VMEM((256,), jnp.float32),
        pltpu.SemaphoreType.DMA,
        pltpu.SemaphoreType.DMA,
    ],
)


def _tc_body(res_ref, y_ref, sq_ref, out_ref):
    res = res_ref[...]
    y = y_ref[...]
    loss = jnp.sum(jax.nn.softplus(-y * res)) / B
    regul = jnp.sum(sq_ref[...]) / (B * D)
    out_ref[0, 0] = loss + LMBDA * regul


def kernel(h, t, r, y, ent_re, ent_im, rel_re, rel_im):
    h32 = h.astype(jnp.int32)
    t32 = t.astype(jnp.int32)
    r32 = r.astype(jnp.int32)
    res, sq = _sc_call(h32, t32, r32, ent_re, ent_im, rel_re, rel_im)
    out = pl.pallas_call(
        _tc_body,
        out_shape=jax.ShapeDtypeStruct((1, 1), jnp.float32),
        out_specs=pl.BlockSpec(memory_space=pltpu.SMEM),
    )(res.reshape(128, 128), y.reshape(128, 128), sq.reshape(4, 128))
    return out[0, 0]
